# Initial kernel scaffold; baseline (speedup 1.0000x reference)
import jax
import jax.numpy as jnp
from jax.experimental import pallas as pl


def kernel(input1, input2, target, class1, class2):
    raise NotImplementedError("write your pallas kernel here")


# speedup vs baseline: 15.4798x; 15.4798x over previous
import jax
import jax.numpy as jnp
from jax.experimental import pallas as pl

_B = 4096
_D = 16
_ALPHA = 0.2
_NB = 10
_BR = 256


def _triplet_block(a_ref, b_ref, out_ref):
    step = pl.program_id(0)
    a = a_ref[...]
    b = b_ref[...]
    a2 = jnp.sum(a * a, axis=1, keepdims=True)
    b2 = jnp.sum(b * b, axis=1)[None, :]
    ab = jax.lax.dot_general(a, b, (((1,), (1,)), ((), ())),
                             preferred_element_type=jnp.float32)
    sq = a2 + b2 - 2.0 * ab
    dist = jnp.sqrt(jnp.maximum(sq, 1e-12))
    row = jax.lax.broadcasted_iota(jnp.int32, (_BR, _B), 0) + step * _BR
    col = jax.lax.broadcasted_iota(jnp.int32, (_BR, _B), 1)
    diag = row == col
    pos = jnp.sum(jnp.where(diag, dist, 0.0), axis=1, keepdims=True)
    cost = jnp.maximum(pos - dist + _ALPHA, 0.0)
    cost = jnp.where(diag, 0.0, cost)
    rem = jnp.full((_BR, 1), float(_NB), dtype=jnp.float32)
    acc = jnp.zeros((_BR, 1), dtype=jnp.float32)
    for _ in range(_NB):
        m = jnp.max(cost, axis=1, keepdims=True)
        eq = cost == m
        cnt = jnp.sum(eq.astype(jnp.float32), axis=1, keepdims=True)
        t = jnp.minimum(cnt, rem)
        acc += m * t
        rem -= t
        cost = jnp.where(eq, -1.0, cost)
    partial = (jnp.sum(acc) * (1.0 / (_B * _NB))).reshape(1, 1)

    @pl.when(step == 0)
    def _():
        out_ref[...] = jnp.zeros((1, 1), dtype=jnp.float32)

    out_ref[...] += partial


def kernel(input1, input2, target, class1, class2):
    out = pl.pallas_call(
        _triplet_block,
        grid=(_B // _BR,),
        in_specs=[
            pl.BlockSpec((_BR, _D), lambda i: (i, 0)),
            pl.BlockSpec((_B, _D), lambda i: (0, 0)),
        ],
        out_specs=pl.BlockSpec((1, 1), lambda i: (0, 0)),
        out_shape=jax.ShapeDtypeStruct((1, 1), jnp.float32),
    )(input1, input2)
    return out[0, 0]


# speedup vs baseline: 23.5747x; 1.5229x over previous
import jax
import jax.numpy as jnp
from jax import lax
from jax.experimental import pallas as pl
from jax.experimental.pallas import tpu as pltpu
from jax.experimental.pallas import tpu_sc as plsc

_B = 4096
_D = 16
_ALPHA = 0.2
_NB = 10
_BR = 256
_NW = 32
_RPW = _B // _NW
_RBLK = 8
_NBLK = _RPW // _RBLK
_L = 16


def _cost_block(a_ref, b_ref, out_ref):
    step = pl.program_id(0)
    a = a_ref[...]
    b = b_ref[...]
    a2 = jnp.sum(a * a, axis=1, keepdims=True)
    b2 = jnp.sum(b * b, axis=1)[None, :]
    ab = lax.dot_general(a, b, (((1,), (1,)), ((), ())),
                         preferred_element_type=jnp.float32)
    dist = jnp.sqrt(jnp.maximum(a2 + b2 - 2.0 * ab, 1e-12))
    row = lax.broadcasted_iota(jnp.int32, (_BR, _B), 0) + step * _BR
    col = lax.broadcasted_iota(jnp.int32, (_BR, _B), 1)
    diag = row == col
    pos = jnp.sum(jnp.where(diag, dist, 0.0), axis=1, keepdims=True)
    cost = jnp.maximum(pos - dist + _ALPHA, 0.0)
    out_ref[...] = jnp.where(diag, 0.0, cost)


def _sc_topk(cost_hbm, out_hbm, buf0, buf1, acc_v, sem0, sem1):
    wid = lax.axis_index("s") * 2 + lax.axis_index("c")
    r0 = wid * _RPW
    bufs = (buf0, buf1)
    sems = (sem0, sem1)
    copies = [None, None]
    copies[0] = pltpu.async_copy(cost_hbm.at[pl.ds(r0, _RBLK)], buf0, sem0)
    lane = lax.broadcasted_iota(jnp.int32, (_L,), 0)
    keep = lane >= (_L - _NB)
    acc = jnp.zeros((_L,), jnp.float32)
    for blk in range(_NBLK):
        if blk + 1 < _NBLK:
            nxt = (blk + 1) % 2
            copies[nxt] = pltpu.async_copy(
                cost_hbm.at[pl.ds(r0 + (blk + 1) * _RBLK, _RBLK)],
                bufs[nxt], sems[nxt])
        copies[blk % 2].wait()
        cur = bufs[blk % 2]

        def body(c, tops):
            new = []
            for r in range(_RBLK):
                g = cur[r, pl.ds(c * _L, _L)]
                g_desc, _ = plsc.sort_key_val(g, g, descending=True)
                u = jnp.maximum(tops[r], g_desc)
                t_asc, _ = plsc.sort_key_val(u, u)
                new.append(t_asc)
            return tuple(new)

        tops = lax.fori_loop(
            0, _B // _L, body,
            tuple(jnp.zeros((_L,), jnp.float32) for _ in range(_RBLK)))
        for r in range(_RBLK):
            acc = acc + jnp.where(keep, tops[r], 0.0)
    acc_v[...] = acc * (1.0 / (_B * _NB))
    pltpu.sync_copy(acc_v, out_hbm.at[wid])


_sc_call = pl.kernel(
    _sc_topk,
    out_type=jax.ShapeDtypeStruct((_NW, _L), jnp.float32),
    mesh=plsc.VectorSubcoreMesh(core_axis_name="c", subcore_axis_name="s"),
    scratch_types=[
        pltpu.VMEM((_RBLK, _B), jnp.float32),
        pltpu.VMEM((_RBLK, _B), jnp.float32),
        pltpu.VMEM((_L,), jnp.float32),
        pltpu.SemaphoreType.DMA,
        pltpu.SemaphoreType.DMA,
    ],
    compiler_params=pltpu.CompilerParams(needs_layout_passes=False),
)


def kernel(input1, input2, target, class1, class2):
    cost = pl.pallas_call(
        _cost_block,
        grid=(_B // _BR,),
        in_specs=[
            pl.BlockSpec((_BR, _D), lambda i: (i, 0)),
            pl.BlockSpec((_B, _D), lambda i: (0, 0)),
        ],
        out_specs=pl.BlockSpec((_BR, _B), lambda i: (i, 0)),
        out_shape=jax.ShapeDtypeStruct((_B, _B), jnp.float32),
    )(input1, input2)
    parts = _sc_call(cost)
    return jnp.sum(parts)


# speedup vs baseline: 26.2612x; 1.1140x over previous
import jax
import jax.numpy as jnp
from jax import lax
from jax.experimental import pallas as pl
from jax.experimental.pallas import tpu as pltpu
from jax.experimental.pallas import tpu_sc as plsc

_B = 4096
_D = 16
_ALPHA = 0.2
_NB = 10
_BR = 256
_NCHUNK = 4
_RC = _B // _NCHUNK
_NW = 32
_RPW = _RC // _NW
_RBLK = 8
_NBLK = _RPW // _RBLK
_L = 16
_NEG = -1e30


def _make_score_block(chunk):
    def _score_block(a_ref, b_ref, out_ref):
        step = pl.program_id(0) + chunk * (_RC // _BR)
        a = a_ref[...]
        b = b_ref[...]
        ha = 0.5 * jnp.sum(a * a, axis=1, keepdims=True)
        hb = 0.5 * jnp.sum(b * b, axis=1)[None, :]
        ab = lax.dot_general(a, b, (((1,), (1,)), ((), ())),
                             preferred_element_type=jnp.float32)
        m = ab - ha - hb
        row = lax.broadcasted_iota(jnp.int32, (_BR, _B), 0) + step * _BR
        col = lax.broadcasted_iota(jnp.int32, (_BR, _B), 1)
        out_ref[...] = jnp.where(row == col, _NEG, m)
    return _score_block


def _sc_top16(m_hbm, out_hbm, buf0, buf1, obuf, sem0, sem1):
    wid = lax.axis_index("s") * 2 + lax.axis_index("c")
    r0 = wid * _RPW
    bufs = (buf0, buf1)
    sems = (sem0, sem1)
    copies = [None, None]
    copies[0] = pltpu.async_copy(m_hbm.at[pl.ds(r0, _RBLK)], buf0, sem0)
    for blk in range(_NBLK):
        if blk + 1 < _NBLK:
            nxt = (blk + 1) % 2
            copies[nxt] = pltpu.async_copy(
                m_hbm.at[pl.ds(r0 + (blk + 1) * _RBLK, _RBLK)],
                bufs[nxt], sems[nxt])
        copies[blk % 2].wait()
        cur = bufs[blk % 2]

        def body(c, tops):
            new = []
            for r in range(_RBLK):
                g = cur[r, pl.ds(c * _L, _L)]
                g_desc, _ = plsc.sort_key_val(g, g, descending=True)
                u = jnp.maximum(tops[r], g_desc)
                t_asc, _ = plsc.sort_key_val(u, u)
                new.append(t_asc)
            return tuple(new)

        tops = lax.fori_loop(
            0, _B // _L, body,
            tuple(jnp.full((_L,), _NEG, jnp.float32) for _ in range(_RBLK)))
        for r in range(_RBLK):
            obuf[r, :] = tops[r]
        pltpu.sync_copy(obuf, out_hbm.at[pl.ds(r0 + blk * _RBLK, _RBLK)])


_sc_call = pl.kernel(
    _sc_top16,
    out_type=jax.ShapeDtypeStruct((_RC, _L), jnp.float32),
    mesh=plsc.VectorSubcoreMesh(core_axis_name="c", subcore_axis_name="s"),
    scratch_types=[
        pltpu.VMEM((_RBLK, _B), jnp.float32),
        pltpu.VMEM((_RBLK, _B), jnp.float32),
        pltpu.VMEM((_RBLK, _L), jnp.float32),
        pltpu.SemaphoreType.DMA,
        pltpu.SemaphoreType.DMA,
    ],
    compiler_params=pltpu.CompilerParams(needs_layout_passes=False),
)


def _finish_block(sel_ref, a_ref, b_ref, out_ref):
    sel = sel_ref[...]
    a = a_ref[...]
    b = b_ref[...]
    diff = a - b
    pos2 = jnp.sum(diff * diff, axis=1, keepdims=True)
    pos = jnp.sqrt(jnp.maximum(pos2, 1e-12))
    d = jnp.sqrt(jnp.maximum(-2.0 * sel, 1e-12))
    cost = jnp.maximum(pos - d + _ALPHA, 0.0)
    lanecol = lax.broadcasted_iota(jnp.int32, (_B, _L), 1)
    kept = jnp.where(lanecol >= (_L - _NB), cost, 0.0)
    out_ref[...] = (jnp.sum(kept) * (1.0 / (_B * _NB))).reshape(1, 1)


def kernel(input1, input2, target, class1, class2):
    sels = []
    for k in range(_NCHUNK):
        m_chunk = pl.pallas_call(
            _make_score_block(k),
            grid=(_RC // _BR,),
            in_specs=[
                pl.BlockSpec((_BR, _D),
                             lambda i, k=k: (i + k * (_RC // _BR), 0)),
                pl.BlockSpec((_B, _D), lambda i: (0, 0)),
            ],
            out_specs=pl.BlockSpec((_BR, _B), lambda i: (i, 0)),
            out_shape=jax.ShapeDtypeStruct((_RC, _B), jnp.float32),
        )(input1, input2)
        sels.append(_sc_call(m_chunk))
    sel = jnp.concatenate(sels, axis=0)
    out = pl.pallas_call(
        _finish_block,
        out_shape=jax.ShapeDtypeStruct((1, 1), jnp.float32),
    )(sel, input1, input2)
    return out[0, 0]


# speedup vs baseline: 29.2352x; 1.1132x over previous
import jax
import jax.numpy as jnp
from jax import lax
from jax.experimental import pallas as pl
from jax.experimental.pallas import tpu as pltpu
from jax.experimental.pallas import tpu_sc as plsc

_B = 4096
_D = 16
_ALPHA = 0.2
_NB = 10
_BR = 256
_NCHUNK = 1
_RC = _B // _NCHUNK
_NW = 32
_RPW = _RC // _NW
_RBLK = 8
_NBLK = _RPW // _RBLK
_L = 16
_NEG = -1e30


def _make_score_block(chunk):
    def _score_block(a_ref, b_ref, out_ref):
        step = pl.program_id(0) + chunk * (_RC // _BR)
        a = a_ref[...]
        b = b_ref[...]
        ha = 0.5 * jnp.sum(a * a, axis=1, keepdims=True)
        hb = 0.5 * jnp.sum(b * b, axis=1)[None, :]
        ab = lax.dot_general(a, b, (((1,), (1,)), ((), ())),
                             preferred_element_type=jnp.float32)
        m = ab - ha - hb
        row = lax.broadcasted_iota(jnp.int32, (_BR, _B), 0) + step * _BR
        col = lax.broadcasted_iota(jnp.int32, (_BR, _B), 1)
        out_ref[...] = jnp.where(row == col, _NEG, m)
    return _score_block


def _sc_top16(m_hbm, out_hbm, buf0, buf1, obuf, sem0, sem1):
    wid = lax.axis_index("s") * 2 + lax.axis_index("c")
    r0 = wid * _RPW
    bufs = (buf0, buf1)
    sems = (sem0, sem1)
    copies = [None, None]
    copies[0] = pltpu.async_copy(m_hbm.at[pl.ds(r0, _RBLK)], buf0, sem0)
    for blk in range(_NBLK):
        if blk + 1 < _NBLK:
            nxt = (blk + 1) % 2
            copies[nxt] = pltpu.async_copy(
                m_hbm.at[pl.ds(r0 + (blk + 1) * _RBLK, _RBLK)],
                bufs[nxt], sems[nxt])
        copies[blk % 2].wait()
        cur = bufs[blk % 2]

        def body(c, tops):
            new = []
            for r in range(_RBLK):
                g = cur[r, pl.ds(c * _L, _L)]
                g_desc, _ = plsc.sort_key_val(g, g, descending=True)
                u = jnp.maximum(tops[r], g_desc)
                t_asc, _ = plsc.sort_key_val(u, u)
                new.append(t_asc)
            return tuple(new)

        tops = lax.fori_loop(
            0, _B // _L, body,
            tuple(jnp.full((_L,), _NEG, jnp.float32) for _ in range(_RBLK)))
        for r in range(_RBLK):
            obuf[r, :] = tops[r]
        pltpu.sync_copy(obuf, out_hbm.at[pl.ds(r0 + blk * _RBLK, _RBLK)])


_sc_call = pl.kernel(
    _sc_top16,
    out_type=jax.ShapeDtypeStruct((_RC, _L), jnp.float32),
    mesh=plsc.VectorSubcoreMesh(core_axis_name="c", subcore_axis_name="s"),
    scratch_types=[
        pltpu.VMEM((_RBLK, _B), jnp.float32),
        pltpu.VMEM((_RBLK, _B), jnp.float32),
        pltpu.VMEM((_RBLK, _L), jnp.float32),
        pltpu.SemaphoreType.DMA,
        pltpu.SemaphoreType.DMA,
    ],
    compiler_params=pltpu.CompilerParams(needs_layout_passes=False),
)


def _finish_block(sel_ref, a_ref, b_ref, out_ref):
    sel = sel_ref[...]
    a = a_ref[...]
    b = b_ref[...]
    diff = a - b
    pos2 = jnp.sum(diff * diff, axis=1, keepdims=True)
    pos = jnp.sqrt(jnp.maximum(pos2, 1e-12))
    d = jnp.sqrt(jnp.maximum(-2.0 * sel, 1e-12))
    cost = jnp.maximum(pos - d + _ALPHA, 0.0)
    lanecol = lax.broadcasted_iota(jnp.int32, (_B, _L), 1)
    kept = jnp.where(lanecol >= (_L - _NB), cost, 0.0)
    out_ref[...] = (jnp.sum(kept) * (1.0 / (_B * _NB))).reshape(1, 1)


def kernel(input1, input2, target, class1, class2):
    sels = []
    for k in range(_NCHUNK):
        m_chunk = pl.pallas_call(
            _make_score_block(k),
            grid=(_RC // _BR,),
            in_specs=[
                pl.BlockSpec((_BR, _D),
                             lambda i, k=k: (i + k * (_RC // _BR), 0)),
                pl.BlockSpec((_B, _D), lambda i: (0, 0)),
            ],
            out_specs=pl.BlockSpec((_BR, _B), lambda i: (i, 0)),
            out_shape=jax.ShapeDtypeStruct((_RC, _B), jnp.float32),
        )(input1, input2)
        sels.append(_sc_call(m_chunk))
    sel = jnp.concatenate(sels, axis=0)
    out = pl.pallas_call(
        _finish_block,
        out_shape=jax.ShapeDtypeStruct((1, 1), jnp.float32),
    )(sel, input1, input2)
    return out[0, 0]
